# baseline (speedup 1.0000x reference)
import functools

import jax
import jax.numpy as jnp
from jax import lax
from jax.experimental import pallas as pl
from jax.experimental.pallas import tpu as pltpu
from jax.experimental.pallas import tpu_sc as plsc

NC = 2
NS = 16
NW = NC * NS
CHUNK = 128


def _make_sc_gather(B, D):
    nchunk = B // (NW * CHUNK)
    mesh = plsc.VectorSubcoreMesh(core_axis_name="c", subcore_axis_name="s")

    @functools.partial(
        pl.kernel,
        mesh=mesh,
        compiler_params=pltpu.CompilerParams(use_tc_tiling_on_sc=False),
        out_type=[
            jax.ShapeDtypeStruct((B // CHUNK, CHUNK, D), jnp.float32),
            jax.ShapeDtypeStruct((B // CHUNK, CHUNK, D), jnp.float32),
        ],
        scratch_types=[
            pltpu.VMEM((nchunk, CHUNK), jnp.int32),
            pltpu.VMEM((nchunk, CHUNK), jnp.int32),
            pltpu.VMEM((nchunk, CHUNK, D), jnp.float32),
            pltpu.VMEM((nchunk, CHUNK, D), jnp.float32),
            pltpu.SemaphoreType.DMA,
            pltpu.SemaphoreType.DMA,
        ],
    )
    def sc_gather(u_hbm, i_hbm, ut_hbm, it_hbm, ue_hbm, ie_hbm,
                  idx_u, idx_i, rows_u, rows_i, sem_u, sem_i):
        wid = lax.axis_index("s") * NC + lax.axis_index("c")
        base = wid * nchunk
        pltpu.sync_copy(u_hbm.at[pl.ds(base, nchunk)], idx_u)
        pltpu.sync_copy(i_hbm.at[pl.ds(base, nchunk)], idx_i)
        copies = []
        for j in range(nchunk):
            copies.append(
                pltpu.async_copy(ut_hbm.at[idx_u.at[j]], rows_u.at[j], sem_u))
            copies.append(
                pltpu.async_copy(it_hbm.at[idx_i.at[j]], rows_i.at[j], sem_i))
        for c in copies:
            c.wait()
        pltpu.sync_copy(rows_u, ue_hbm.at[pl.ds(base, nchunk)])
        pltpu.sync_copy(rows_i, ie_hbm.at[pl.ds(base, nchunk)])

    return sc_gather


def _mlp_body(ue_ref, ie_ref, w1_ref, b1_ref, w2_ref, b2_ref,
              w3_ref, b3_ref, wp_ref, bp_ref, out_ref):
    x = jnp.concatenate([ue_ref[...], ie_ref[...]], axis=1)
    dn = (((1,), (1,)), ((), ()))
    h = lax.dot_general(x, w1_ref[...], dn, preferred_element_type=jnp.float32)
    h = jnp.maximum(h + b1_ref[...], 0.0)
    h = lax.dot_general(h, w2_ref[...], dn, preferred_element_type=jnp.float32)
    h = jnp.maximum(h + b2_ref[...], 0.0)
    h = lax.dot_general(h, w3_ref[...], dn, preferred_element_type=jnp.float32)
    h = jnp.maximum(h + b3_ref[...], 0.0)
    out_ref[...] = jnp.sum(h * wp_ref[...], axis=1) + bp_ref[0]


def _mlp_call(ue, ie, W1, b1, W2, b2, W3, b3, Wp, bp, block_b):
    B = ue.shape[0]
    grid = (B // block_b,)
    full = lambda shape: pl.BlockSpec(shape, lambda ib: (0,) * len(shape))
    return pl.pallas_call(
        _mlp_body,
        grid=grid,
        in_specs=[
            pl.BlockSpec((block_b, ue.shape[1]), lambda ib: (ib, 0)),
            pl.BlockSpec((block_b, ie.shape[1]), lambda ib: (ib, 0)),
            full(W1.shape),
            full((1, b1.shape[0])),
            full(W2.shape),
            full((1, b2.shape[0])),
            full(W3.shape),
            full((1, b3.shape[0])),
            full(Wp.shape),
            pl.BlockSpec(memory_space=pltpu.SMEM),
        ],
        out_specs=pl.BlockSpec((block_b,), lambda ib: (ib,)),
        out_shape=jax.ShapeDtypeStruct((B,), jnp.float32),
    )(ue, ie, W1, b1.reshape(1, -1), W2, b2.reshape(1, -1),
      W3, b3.reshape(1, -1), Wp, bp)


def kernel(u, i, user_table, item_table, W1, b1, W2, b2, W3, b3, Wp, bp):
    B = u.shape[0]
    D = user_table.shape[1]
    u2 = u.astype(jnp.int32).reshape(B // CHUNK, CHUNK)
    i2 = i.astype(jnp.int32).reshape(B // CHUNK, CHUNK)
    ue3, ie3 = _make_sc_gather(B, D)(u2, i2, user_table, item_table)
    ue = ue3.reshape(B, D)
    ie = ie3.reshape(B, D)
    return _mlp_call(ue, ie, W1, b1, W2, b2, W3, b3, Wp, bp, block_b=2048)
